# baseline (speedup 1.0000x reference)
import functools

import jax
import jax.numpy as jnp
from jax import lax
from jax.experimental import pallas as pl
from jax.experimental.pallas import tpu as pltpu
from jax.experimental.pallas import tpu_sc as plsc

_THRESH_AFF = 0.65
_THRESH_REG = 0.6
_LAMBDA = 2.0

_NC, _NS, _L = 2, 16, 16
_NW = _NC * _NS
_B = 16
_H = _W = 384
_N = _H * _W
_HALF = _N // 2
_SC_B = 8
_TC_B = _B - _SC_B
_ROWS = 24
_RCH = _ROWS * _W
_NCH_STATS = (_H // 4) // _ROWS
_CH = 8192
_NCH_TOPK = _N // _CH

_mesh = plsc.VectorSubcoreMesh(
    core_axis_name="c", subcore_axis_name="s", num_cores=_NC, num_subcores=_NS
)


@functools.partial(
    pl.kernel,
    out_type=jax.ShapeDtypeStruct((_NW, 6 * _L), jnp.float32),
    mesh=_mesh,
    compiler_params=pltpu.CompilerParams(needs_layout_passes=False),
    scratch_types=(
        [pltpu.VMEM((_ROWS, _W), jnp.float32) for _ in range(10)]
        + [
            pltpu.VMEM((6 * _L,), jnp.float32),
            pltpu.SemaphoreType.DMA,
            pltpu.SemaphoreType.DMA,
        ]
    ),
)
def _stats_kernel(rg_hbm, rp_hbm, ag_hbm, ap_hbm, mk_hbm, out_hbm,
                  b0a, b1a, b2a, b3a, b4a, b0b, b1b, b2b, b3b, b4b,
                  accv, sem_a, sem_b):
    c = lax.axis_index("c")
    s = lax.axis_index("s")
    wid = c * _NS + s
    img = wid // 4
    row0 = (wid % 4) * (_H // 4)
    ins = (rg_hbm, rp_hbm, ag_hbm, ap_hbm, mk_hbm)
    bufs = ((b0a, b1a, b2a, b3a, b4a), (b0b, b1b, b2b, b3b, b4b))
    sems = (sem_a, sem_b)

    def issue(t, slot):
        r = row0 + t * _ROWS
        return [
            pltpu.async_copy(ins[a].at[img, pl.ds(r, _ROWS), :], bufs[slot][a], sems[slot])
            for a in range(5)
        ]

    zerof = jnp.zeros((_L,), jnp.float32)
    zeroi = jnp.zeros((_L,), jnp.int32)
    accs = (zeroi, zerof, zerof, zeroi, zerof, zerof)
    pending = [None, None]
    pending[0] = issue(0, 0)
    for t in range(_NCH_STATS):
        slot = t % 2
        if t + 1 < _NCH_STATS:
            pending[1 - slot] = issue(t + 1, 1 - slot)
        for h in pending[slot]:
            h.wait()
        buf = bufs[slot]

        def body(r, o, accs, buf=buf):
            pcr, psr, tsr, pca, psa, tsa = accs
            rg = buf[0][r, pl.ds(o, _L)]
            rp = buf[1][r, pl.ds(o, _L)]
            ag = buf[2][r, pl.ds(o, _L)]
            ap = buf[3][r, pl.ds(o, _L)]
            mk = buf[4][r, pl.ds(o, _L)]
            one = jnp.float32(1.0)
            zf = jnp.float32(0.0)
            pos_r = rg > _THRESH_REG
            dr = jnp.where(pos_r, jnp.minimum(rp, one), rp) - rg
            lr = dr * dr * mk
            pos_a = ag > _THRESH_AFF
            da = jnp.where(pos_a, jnp.minimum(ap, one), ap) - ag
            la = da * da * mk
            return (
                pcr + plsc.all_reduce_population_count(pos_r),
                psr + jnp.where(pos_r, lr, zf),
                tsr + lr,
                pca + plsc.all_reduce_population_count(pos_a),
                psa + jnp.where(pos_a, la, zf),
                tsa + la,
            )

        def row_body(r, accs, body=body):
            def col_body(cj, a, r=r):
                for u in range(4):
                    a = body(r, cj * 4 * _L + u * _L, a)
                return a

            return lax.fori_loop(0, _W // (4 * _L), col_body, accs)

        accs = lax.fori_loop(0, _ROWS, row_body, accs)
    for i, a in enumerate(accs):
        accv[pl.ds(i * _L, _L)] = a.astype(jnp.float32)
    pltpu.sync_copy(accv, out_hbm.at[wid])


@functools.partial(
    pl.kernel,
    out_type=jax.ShapeDtypeStruct((_NW * 3 * _L,), jnp.float32),
    mesh=_mesh,
    compiler_params=pltpu.CompilerParams(needs_layout_passes=False),
    scratch_types=(
        [pltpu.VMEM((_CH,), jnp.float32) for _ in range(6)]
        + [
            pltpu.VMEM((3 * _L,), jnp.float32),
            pltpu.SemaphoreType.DMA,
            pltpu.SemaphoreType.DMA,
        ]
    ),
)
def _topk_kernel(gt_hbm, pr_hbm, mk_hbm, kk_hbm, out_hbm,
                 g_a, p_a, m_a, g_b, p_b, m_b, vout, sem_a, sem_b):
    lt = lax.axis_index("c")
    img = lax.axis_index("s")
    w = lt * _NS + img
    gbase = lt * (_B * _N) + img * _N
    mbase = img * _N
    thr = jnp.where(lt == 0, jnp.float32(_THRESH_REG), jnp.float32(_THRESH_AFF))
    pltpu.sync_copy(kk_hbm.at[pl.ds(w * _L, _L)], vout.at[pl.ds(0, _L)])
    kv = vout[pl.ds(0, _L)]
    bufs = ((g_a, p_a, m_a), (g_b, p_b, m_b))
    sems = (sem_a, sem_b)

    def sweep(cand, want_final):
        def issue(t, slot):
            off = t * _CH
            return [
                pltpu.async_copy(gt_hbm.at[pl.ds(gbase + off, _CH)], bufs[slot][0], sems[slot]),
                pltpu.async_copy(pr_hbm.at[pl.ds(gbase + off, _CH)], bufs[slot][1], sems[slot]),
                pltpu.async_copy(mk_hbm.at[pl.ds(mbase + off, _CH)], bufs[slot][2], sems[slot]),
            ]

        zero = jnp.zeros((_L,), jnp.float32)
        accs = (zero, zero, zero)
        pending = [None, None]
        pending[0] = issue(0, 0)
        for t in range(_NCH_TOPK):
            slot = t % 2
            if t + 1 < _NCH_TOPK:
                pending[1 - slot] = issue(t + 1, 1 - slot)
            for h in pending[slot]:
                h.wait()
            buf = bufs[slot]

            def body(j, accs, buf=buf):
                cge, cgt, sgt = accs
                o = j * _L
                gt = buf[0][pl.ds(o, _L)]
                pr = buf[1][pl.ds(o, _L)]
                mk = buf[2][pl.ds(o, _L)]
                pos = gt > thr
                pr2 = jnp.where(pos & (pr > 1.0), 1.0, pr)
                d = pr2 - gt
                v = jnp.where(pos, jnp.float32(-1.0), d * d * mk)
                cge = cge + plsc.all_reduce_population_count(v >= cand).astype(jnp.float32)
                if want_final:
                    cgt = cgt + plsc.all_reduce_population_count(v > cand).astype(jnp.float32)
                    sgt = sgt + jnp.where(v > cand, v, jnp.float32(0.0))
                return (cge, cgt, sgt)

            accs = lax.fori_loop(0, _CH // _L, body, accs)
        return accs

    def phase_a(i, cand):
        cge, _, _ = sweep(cand, False)
        return jnp.where(cge >= kv, cand, cand * jnp.float32(1.0 / 65536.0))

    def phase_b(i, cand):
        c2 = cand * jnp.float32(2.0)
        cge, _, _ = sweep(c2, False)
        return jnp.where(cge >= kv, c2, cand)

    def phase_c(i, lohi):
        lo, hi = lohi
        mid = (lo + hi) * jnp.float32(0.5)
        cge, _, _ = sweep(mid, False)
        acc = cge >= kv
        return (jnp.where(acc, mid, lo), jnp.where(acc, hi, mid))

    start = jnp.full((_L,), 2.0**124, jnp.float32)
    cand = lax.fori_loop(0, 17, phase_a, start)
    cand = lax.fori_loop(0, 16, phase_b, cand)
    lo, hi = lax.fori_loop(0, 30, phase_c, (cand, cand * jnp.float32(2.0)))
    _, cgt, sgt = sweep(lo, True)
    vout[pl.ds(0, _L)] = sgt
    vout[pl.ds(_L, _L)] = cgt
    vout[pl.ds(2 * _L, _L)] = lo
    pltpu.sync_copy(vout, out_hbm.at[pl.ds(w * 3 * _L, 3 * _L)])


@functools.partial(
    pl.pallas_call,
    grid=(_TC_B,),
    in_specs=[pl.BlockSpec((1, _H, _W), lambda i: (i, 0, 0))] * 5,
    out_specs=pl.BlockSpec((1, 1, 8), lambda i: (i, 0, 0), memory_space=pltpu.SMEM),
    out_shape=jax.ShapeDtypeStruct((_TC_B, 1, 8), jnp.float32),
)
def _tc_stats(rg_ref, rp_ref, ag_ref, ap_ref, mk_ref, out_ref):
    rg = rg_ref[0]
    rp = rp_ref[0]
    ag = ag_ref[0]
    ap = ap_ref[0]
    mk = mk_ref[0]
    one = jnp.float32(1.0)
    zf = jnp.float32(0.0)
    pos_r = rg > _THRESH_REG
    dr = jnp.where(pos_r, jnp.minimum(rp, one), rp) - rg
    lr = dr * dr * mk
    pos_a = ag > _THRESH_AFF
    da = jnp.where(pos_a, jnp.minimum(ap, one), ap) - ag
    la = da * da * mk
    out_ref[0, 0, 0] = jnp.sum(pos_r.astype(jnp.float32))
    out_ref[0, 0, 1] = jnp.sum(jnp.where(pos_r, lr, zf))
    out_ref[0, 0, 2] = jnp.sum(lr)
    out_ref[0, 0, 3] = jnp.sum(pos_a.astype(jnp.float32))
    out_ref[0, 0, 4] = jnp.sum(jnp.where(pos_a, la, zf))
    out_ref[0, 0, 5] = jnp.sum(la)
    out_ref[0, 0, 6] = zf
    out_ref[0, 0, 7] = zf


def _combine(pos, psum, nsum, topk_mean):
    npix = jnp.float32(_N)
    neg = npix - pos
    posi = psum / jnp.maximum(pos, 1.0)
    nega_mean = jnp.where(neg > 0, nsum / jnp.maximum(neg, 1.0), 0.0)
    nega = jnp.where(neg < 3.0 * pos, nega_mean, topk_mean)
    return jnp.where(pos > 0, posi + nega, topk_mean)


def kernel(region_score_gt, affinity_score_gt, region_score_pred,
           affinity_score_pred, mask):
    stats = _stats_kernel(region_score_gt, region_score_pred,
                          affinity_score_gt, affinity_score_pred, mask)
    tc3 = _tc_stats(region_score_gt[_SC_B:], region_score_pred[_SC_B:],
                   affinity_score_gt[_SC_B:], affinity_score_pred[_SC_B:],
                   mask[_SC_B:])
    tc = tc3[:, 0, :]
    per_w = stats.reshape(_NW, 6, _L)
    counts = per_w[:, (0, 3), 0]
    sums = per_w[:, (1, 2, 4, 5), :].sum(-1)
    sc_c = counts.reshape(_SC_B, 4, 2).sum(1)
    sc_s = sums.reshape(_SC_B, 4, 4).sum(1)
    pos_r = jnp.concatenate([sc_c[:, 0], tc[:, 0]])
    pos_a = jnp.concatenate([sc_c[:, 1], tc[:, 3]])
    psum_r = jnp.concatenate([sc_s[:, 0], tc[:, 1]])
    tsum_r = jnp.concatenate([sc_s[:, 1], tc[:, 2]])
    psum_a = jnp.concatenate([sc_s[:, 2], tc[:, 4]])
    tsum_a = jnp.concatenate([sc_s[:, 3], tc[:, 5]])
    nsum_r = tsum_r - psum_r
    nsum_a = tsum_a - psum_a

    npix = jnp.float32(_N)
    need = jnp.any((pos_r == 0) | (npix - pos_r >= 3.0 * pos_r)) | jnp.any(
        (pos_a == 0) | (npix - pos_a >= 3.0 * pos_a)
    )

    def rare_branch():
        rgf = region_score_gt.reshape(-1)
        agf = affinity_score_gt.reshape(-1)
        rpf = region_score_pred.reshape(-1)
        apf = affinity_score_pred.reshape(-1)
        mkf = mask.reshape(-1)
        gts = jnp.concatenate([rgf, agf])
        prs = jnp.concatenate([rpf, apf])
        k_r = jnp.where(pos_r > 0, 3.0 * pos_r, 500.0)
        k_a = jnp.where(pos_a > 0, 3.0 * pos_a, 500.0)
        kk2 = jnp.stack([k_r, k_a])
        kk = jnp.broadcast_to(kk2[:, :, None], (2, _B, _L)).reshape(-1)
        out = _topk_kernel(gts, prs, mkf, kk)
        o = out.reshape(2, _B, 3, _L)
        sgt = o[:, :, 0, :].sum(-1)
        cgt = o[:, :, 1, 0]
        tval = o[:, :, 2, 0]
        return (sgt + (kk2 - cgt) * tval) / kk2

    topk_means = lax.cond(need, rare_branch, lambda: jnp.zeros((2, _B), jnp.float32))

    contrib_r = _combine(pos_r, psum_r, nsum_r, topk_means[0])
    contrib_a = _combine(pos_a, psum_a, nsum_a, topk_means[1])
    char_loss = jnp.sum(contrib_r)
    affi_loss = jnp.sum(contrib_a)
    return _LAMBDA * char_loss / _B + affi_loss / _B


# speedup vs baseline: 1.3700x; 1.3700x over previous
import functools

import jax
import jax.numpy as jnp
from jax import lax
from jax.experimental import pallas as pl
from jax.experimental.pallas import tpu as pltpu
from jax.experimental.pallas import tpu_sc as plsc

_THRESH_AFF = 0.65
_THRESH_REG = 0.6
_LAMBDA = 2.0

_NC, _NS, _L = 2, 16, 16
_NW = _NC * _NS
_B = 16
_H = _W = 384
_N = _H * _W
_HALF = _N // 2
_SC_B = 8
_TC_B = _B - _SC_B
_ROWS = 24
_RCH = _ROWS * _W
_NCH_STATS = (_H // 4) // _ROWS
_CH = 8192
_NCH_TOPK = _N // _CH

_mesh = plsc.VectorSubcoreMesh(
    core_axis_name="c", subcore_axis_name="s", num_cores=_NC, num_subcores=_NS
)


@functools.partial(
    pl.kernel,
    out_type=jax.ShapeDtypeStruct((_NW, 6 * _L), jnp.float32),
    mesh=_mesh,
    compiler_params=pltpu.CompilerParams(needs_layout_passes=False),
    scratch_types=(
        [pltpu.VMEM((_ROWS, _W), jnp.float32) for _ in range(10)]
        + [
            pltpu.VMEM((6 * _L,), jnp.float32),
            pltpu.SemaphoreType.DMA,
            pltpu.SemaphoreType.DMA,
        ]
    ),
)
def _stats_kernel(rg_hbm, rp_hbm, ag_hbm, ap_hbm, mk_hbm, out_hbm,
                  b0a, b1a, b2a, b3a, b4a, b0b, b1b, b2b, b3b, b4b,
                  accv, sem_a, sem_b):
    c = lax.axis_index("c")
    s = lax.axis_index("s")
    wid = c * _NS + s
    img = wid // 4
    row0 = (wid % 4) * (_H // 4)
    ins = (rg_hbm, rp_hbm, ag_hbm, ap_hbm, mk_hbm)
    bufs = ((b0a, b1a, b2a, b3a, b4a), (b0b, b1b, b2b, b3b, b4b))
    sems = (sem_a, sem_b)

    def issue(t, slot):
        r = row0 + t * _ROWS
        return [
            pltpu.async_copy(ins[a].at[img, pl.ds(r, _ROWS), :], bufs[slot][a], sems[slot])
            for a in range(5)
        ]

    zerof = jnp.zeros((_L,), jnp.float32)
    zeroi = jnp.zeros((_L,), jnp.int32)
    accs = (zeroi, zerof, zerof, zeroi, zerof, zerof)
    pending = [None, None]
    pending[0] = issue(0, 0)
    for t in range(_NCH_STATS):
        slot = t % 2
        if t + 1 < _NCH_STATS:
            pending[1 - slot] = issue(t + 1, 1 - slot)
        for h in pending[slot]:
            h.wait()
        buf = bufs[slot]

        def body(r, o, accs, buf=buf):
            pcr, psr, tsr, pca, psa, tsa = accs
            rg = buf[0][r, pl.ds(o, _L)]
            rp = buf[1][r, pl.ds(o, _L)]
            ag = buf[2][r, pl.ds(o, _L)]
            ap = buf[3][r, pl.ds(o, _L)]
            mk = buf[4][r, pl.ds(o, _L)]
            one = jnp.float32(1.0)
            zf = jnp.float32(0.0)
            pos_r = rg > _THRESH_REG
            dr = jnp.where(pos_r, jnp.minimum(rp, one), rp) - rg
            lr = dr * dr * mk
            pos_a = ag > _THRESH_AFF
            da = jnp.where(pos_a, jnp.minimum(ap, one), ap) - ag
            la = da * da * mk
            return (
                pcr + plsc.all_reduce_population_count(pos_r),
                psr + jnp.where(pos_r, lr, zf),
                tsr + lr,
                pca + plsc.all_reduce_population_count(pos_a),
                psa + jnp.where(pos_a, la, zf),
                tsa + la,
            )

        def row_body(r, accs, body=body):
            def col_body(cj, a, r=r):
                for u in range(4):
                    a = body(r, cj * 4 * _L + u * _L, a)
                return a

            return lax.fori_loop(0, _W // (4 * _L), col_body, accs)

        accs = lax.fori_loop(0, _ROWS, row_body, accs)
    for i, a in enumerate(accs):
        accv[pl.ds(i * _L, _L)] = a.astype(jnp.float32)
    pltpu.sync_copy(accv, out_hbm.at[wid])


@functools.partial(
    pl.kernel,
    out_type=jax.ShapeDtypeStruct((_NW * 3 * _L,), jnp.float32),
    mesh=_mesh,
    compiler_params=pltpu.CompilerParams(needs_layout_passes=False),
    scratch_types=(
        [pltpu.VMEM((_CH,), jnp.float32) for _ in range(6)]
        + [
            pltpu.VMEM((3 * _L,), jnp.float32),
            pltpu.SemaphoreType.DMA,
            pltpu.SemaphoreType.DMA,
        ]
    ),
)
def _topk_kernel(gt_hbm, pr_hbm, mk_hbm, kk_hbm, out_hbm,
                 g_a, p_a, m_a, g_b, p_b, m_b, vout, sem_a, sem_b):
    lt = lax.axis_index("c")
    img = lax.axis_index("s")
    w = lt * _NS + img
    gbase = lt * (_B * _N) + img * _N
    mbase = img * _N
    thr = jnp.where(lt == 0, jnp.float32(_THRESH_REG), jnp.float32(_THRESH_AFF))
    pltpu.sync_copy(kk_hbm.at[pl.ds(w * _L, _L)], vout.at[pl.ds(0, _L)])
    kv = vout[pl.ds(0, _L)]
    bufs = ((g_a, p_a, m_a), (g_b, p_b, m_b))
    sems = (sem_a, sem_b)

    def sweep(cand, want_final):
        def issue(t, slot):
            off = t * _CH
            return [
                pltpu.async_copy(gt_hbm.at[pl.ds(gbase + off, _CH)], bufs[slot][0], sems[slot]),
                pltpu.async_copy(pr_hbm.at[pl.ds(gbase + off, _CH)], bufs[slot][1], sems[slot]),
                pltpu.async_copy(mk_hbm.at[pl.ds(mbase + off, _CH)], bufs[slot][2], sems[slot]),
            ]

        zero = jnp.zeros((_L,), jnp.float32)
        accs = (zero, zero, zero)
        pending = [None, None]
        pending[0] = issue(0, 0)
        for t in range(_NCH_TOPK):
            slot = t % 2
            if t + 1 < _NCH_TOPK:
                pending[1 - slot] = issue(t + 1, 1 - slot)
            for h in pending[slot]:
                h.wait()
            buf = bufs[slot]

            def body(j, accs, buf=buf):
                cge, cgt, sgt = accs
                o = j * _L
                gt = buf[0][pl.ds(o, _L)]
                pr = buf[1][pl.ds(o, _L)]
                mk = buf[2][pl.ds(o, _L)]
                pos = gt > thr
                pr2 = jnp.where(pos & (pr > 1.0), 1.0, pr)
                d = pr2 - gt
                v = jnp.where(pos, jnp.float32(-1.0), d * d * mk)
                cge = cge + plsc.all_reduce_population_count(v >= cand).astype(jnp.float32)
                if want_final:
                    cgt = cgt + plsc.all_reduce_population_count(v > cand).astype(jnp.float32)
                    sgt = sgt + jnp.where(v > cand, v, jnp.float32(0.0))
                return (cge, cgt, sgt)

            accs = lax.fori_loop(0, _CH // _L, body, accs)
        return accs

    def phase_a(i, cand):
        cge, _, _ = sweep(cand, False)
        return jnp.where(cge >= kv, cand, cand * jnp.float32(1.0 / 65536.0))

    def phase_b(i, cand):
        c2 = cand * jnp.float32(2.0)
        cge, _, _ = sweep(c2, False)
        return jnp.where(cge >= kv, c2, cand)

    def phase_c(i, lohi):
        lo, hi = lohi
        mid = (lo + hi) * jnp.float32(0.5)
        cge, _, _ = sweep(mid, False)
        acc = cge >= kv
        return (jnp.where(acc, mid, lo), jnp.where(acc, hi, mid))

    start = jnp.full((_L,), 2.0**124, jnp.float32)
    cand = lax.fori_loop(0, 17, phase_a, start)
    cand = lax.fori_loop(0, 16, phase_b, cand)
    lo, hi = lax.fori_loop(0, 30, phase_c, (cand, cand * jnp.float32(2.0)))
    _, cgt, sgt = sweep(lo, True)
    vout[pl.ds(0, _L)] = sgt
    vout[pl.ds(_L, _L)] = cgt
    vout[pl.ds(2 * _L, _L)] = lo
    pltpu.sync_copy(vout, out_hbm.at[pl.ds(w * 3 * _L, 3 * _L)])


@functools.partial(
    pl.pallas_call,
    grid=(_TC_B,),
    in_specs=[pl.BlockSpec((1, _H, _W), lambda i: (i + _SC_B, 0, 0))] * 5,
    out_specs=pl.BlockSpec((1, 1, 8), lambda i: (i, 0, 0), memory_space=pltpu.SMEM),
    out_shape=jax.ShapeDtypeStruct((_TC_B, 1, 8), jnp.float32),
)
def _tc_stats(rg_ref, rp_ref, ag_ref, ap_ref, mk_ref, out_ref):
    rg = rg_ref[0]
    rp = rp_ref[0]
    ag = ag_ref[0]
    ap = ap_ref[0]
    mk = mk_ref[0]
    one = jnp.float32(1.0)
    zf = jnp.float32(0.0)
    pos_r = rg > _THRESH_REG
    dr = jnp.where(pos_r, jnp.minimum(rp, one), rp) - rg
    lr = dr * dr * mk
    pos_a = ag > _THRESH_AFF
    da = jnp.where(pos_a, jnp.minimum(ap, one), ap) - ag
    la = da * da * mk
    out_ref[0, 0, 0] = jnp.sum(pos_r.astype(jnp.float32))
    out_ref[0, 0, 1] = jnp.sum(jnp.where(pos_r, lr, zf))
    out_ref[0, 0, 2] = jnp.sum(lr)
    out_ref[0, 0, 3] = jnp.sum(pos_a.astype(jnp.float32))
    out_ref[0, 0, 4] = jnp.sum(jnp.where(pos_a, la, zf))
    out_ref[0, 0, 5] = jnp.sum(la)
    out_ref[0, 0, 6] = zf
    out_ref[0, 0, 7] = zf


def _combine(pos, psum, nsum, topk_mean):
    npix = jnp.float32(_N)
    neg = npix - pos
    posi = psum / jnp.maximum(pos, 1.0)
    nega_mean = jnp.where(neg > 0, nsum / jnp.maximum(neg, 1.0), 0.0)
    nega = jnp.where(neg < 3.0 * pos, nega_mean, topk_mean)
    return jnp.where(pos > 0, posi + nega, topk_mean)


def kernel(region_score_gt, affinity_score_gt, region_score_pred,
           affinity_score_pred, mask):
    stats = _stats_kernel(region_score_gt, region_score_pred,
                          affinity_score_gt, affinity_score_pred, mask)
    tc3 = _tc_stats(region_score_gt, region_score_pred,
                    affinity_score_gt, affinity_score_pred,
                    mask)
    tc = tc3[:, 0, :]
    per_w = stats.reshape(_NW, 6, _L)
    counts = per_w[:, (0, 3), 0]
    sums = per_w[:, (1, 2, 4, 5), :].sum(-1)
    sc_c = counts.reshape(_SC_B, 4, 2).sum(1)
    sc_s = sums.reshape(_SC_B, 4, 4).sum(1)
    pos_r = jnp.concatenate([sc_c[:, 0], tc[:, 0]])
    pos_a = jnp.concatenate([sc_c[:, 1], tc[:, 3]])
    psum_r = jnp.concatenate([sc_s[:, 0], tc[:, 1]])
    tsum_r = jnp.concatenate([sc_s[:, 1], tc[:, 2]])
    psum_a = jnp.concatenate([sc_s[:, 2], tc[:, 4]])
    tsum_a = jnp.concatenate([sc_s[:, 3], tc[:, 5]])
    nsum_r = tsum_r - psum_r
    nsum_a = tsum_a - psum_a

    npix = jnp.float32(_N)
    need = jnp.any((pos_r == 0) | (npix - pos_r >= 3.0 * pos_r)) | jnp.any(
        (pos_a == 0) | (npix - pos_a >= 3.0 * pos_a)
    )

    def rare_branch():
        rgf = region_score_gt.reshape(-1)
        agf = affinity_score_gt.reshape(-1)
        rpf = region_score_pred.reshape(-1)
        apf = affinity_score_pred.reshape(-1)
        mkf = mask.reshape(-1)
        gts = jnp.concatenate([rgf, agf])
        prs = jnp.concatenate([rpf, apf])
        k_r = jnp.where(pos_r > 0, 3.0 * pos_r, 500.0)
        k_a = jnp.where(pos_a > 0, 3.0 * pos_a, 500.0)
        kk2 = jnp.stack([k_r, k_a])
        kk = jnp.broadcast_to(kk2[:, :, None], (2, _B, _L)).reshape(-1)
        out = _topk_kernel(gts, prs, mkf, kk)
        o = out.reshape(2, _B, 3, _L)
        sgt = o[:, :, 0, :].sum(-1)
        cgt = o[:, :, 1, 0]
        tval = o[:, :, 2, 0]
        return (sgt + (kk2 - cgt) * tval) / kk2

    topk_means = lax.cond(need, rare_branch, lambda: jnp.zeros((2, _B), jnp.float32))

    contrib_r = _combine(pos_r, psum_r, nsum_r, topk_means[0])
    contrib_a = _combine(pos_a, psum_a, nsum_a, topk_means[1])
    char_loss = jnp.sum(contrib_r)
    affi_loss = jnp.sum(contrib_a)
    return _LAMBDA * char_loss / _B + affi_loss / _B


# speedup vs baseline: 1.4354x; 1.0477x over previous
import functools

import jax
import jax.numpy as jnp
from jax import lax
from jax.experimental import pallas as pl
from jax.experimental.pallas import tpu as pltpu
from jax.experimental.pallas import tpu_sc as plsc

_THRESH_AFF = 0.65
_THRESH_REG = 0.6
_LAMBDA = 2.0

_NC, _NS, _L = 2, 16, 16
_NW = _NC * _NS
_B = 16
_H = _W = 384
_N = _H * _W
_HALF = _N // 2
_SC_B = 4
_TC_B = _B - _SC_B
_ROWS = 24
_RCH = _ROWS * _W
_NCH_STATS = (_H // 8) // _ROWS
_CH = 8192
_NCH_TOPK = _N // _CH

_mesh = plsc.VectorSubcoreMesh(
    core_axis_name="c", subcore_axis_name="s", num_cores=_NC, num_subcores=_NS
)


@functools.partial(
    pl.kernel,
    out_type=jax.ShapeDtypeStruct((_NW, 6 * _L), jnp.float32),
    mesh=_mesh,
    compiler_params=pltpu.CompilerParams(needs_layout_passes=False),
    scratch_types=(
        [pltpu.VMEM((_ROWS, _W), jnp.float32) for _ in range(10)]
        + [
            pltpu.VMEM((6 * _L,), jnp.float32),
            pltpu.SemaphoreType.DMA,
            pltpu.SemaphoreType.DMA,
        ]
    ),
)
def _stats_kernel(rg_hbm, rp_hbm, ag_hbm, ap_hbm, mk_hbm, out_hbm,
                  b0a, b1a, b2a, b3a, b4a, b0b, b1b, b2b, b3b, b4b,
                  accv, sem_a, sem_b):
    c = lax.axis_index("c")
    s = lax.axis_index("s")
    wid = c * _NS + s
    img = wid // 8
    row0 = (wid % 8) * (_H // 8)
    ins = (rg_hbm, rp_hbm, ag_hbm, ap_hbm, mk_hbm)
    bufs = ((b0a, b1a, b2a, b3a, b4a), (b0b, b1b, b2b, b3b, b4b))
    sems = (sem_a, sem_b)

    def issue(t, slot):
        r = row0 + t * _ROWS
        return [
            pltpu.async_copy(ins[a].at[img, pl.ds(r, _ROWS), :], bufs[slot][a], sems[slot])
            for a in range(5)
        ]

    zerof = jnp.zeros((_L,), jnp.float32)
    zeroi = jnp.zeros((_L,), jnp.int32)
    accs = (zeroi, zerof, zerof, zeroi, zerof, zerof)
    pending = [None, None]
    pending[0] = issue(0, 0)
    for t in range(_NCH_STATS):
        slot = t % 2
        if t + 1 < _NCH_STATS:
            pending[1 - slot] = issue(t + 1, 1 - slot)
        for h in pending[slot]:
            h.wait()
        buf = bufs[slot]

        def body(r, o, accs, buf=buf):
            pcr, psr, tsr, pca, psa, tsa = accs
            rg = buf[0][r, pl.ds(o, _L)]
            rp = buf[1][r, pl.ds(o, _L)]
            ag = buf[2][r, pl.ds(o, _L)]
            ap = buf[3][r, pl.ds(o, _L)]
            mk = buf[4][r, pl.ds(o, _L)]
            one = jnp.float32(1.0)
            zf = jnp.float32(0.0)
            pos_r = rg > _THRESH_REG
            dr = jnp.where(pos_r, jnp.minimum(rp, one), rp) - rg
            lr = dr * dr * mk
            pos_a = ag > _THRESH_AFF
            da = jnp.where(pos_a, jnp.minimum(ap, one), ap) - ag
            la = da * da * mk
            return (
                pcr + plsc.all_reduce_population_count(pos_r),
                psr + jnp.where(pos_r, lr, zf),
                tsr + lr,
                pca + plsc.all_reduce_population_count(pos_a),
                psa + jnp.where(pos_a, la, zf),
                tsa + la,
            )

        def row_body(r, accs, body=body):
            def col_body(cj, a, r=r):
                for u in range(4):
                    a = body(r, cj * 4 * _L + u * _L, a)
                return a

            return lax.fori_loop(0, _W // (4 * _L), col_body, accs)

        accs = lax.fori_loop(0, _ROWS, row_body, accs)
    for i, a in enumerate(accs):
        accv[pl.ds(i * _L, _L)] = a.astype(jnp.float32)
    pltpu.sync_copy(accv, out_hbm.at[wid])


@functools.partial(
    pl.kernel,
    out_type=jax.ShapeDtypeStruct((_NW * 3 * _L,), jnp.float32),
    mesh=_mesh,
    compiler_params=pltpu.CompilerParams(needs_layout_passes=False),
    scratch_types=(
        [pltpu.VMEM((_CH,), jnp.float32) for _ in range(6)]
        + [
            pltpu.VMEM((3 * _L,), jnp.float32),
            pltpu.SemaphoreType.DMA,
            pltpu.SemaphoreType.DMA,
        ]
    ),
)
def _topk_kernel(gt_hbm, pr_hbm, mk_hbm, kk_hbm, out_hbm,
                 g_a, p_a, m_a, g_b, p_b, m_b, vout, sem_a, sem_b):
    lt = lax.axis_index("c")
    img = lax.axis_index("s")
    w = lt * _NS + img
    gbase = lt * (_B * _N) + img * _N
    mbase = img * _N
    thr = jnp.where(lt == 0, jnp.float32(_THRESH_REG), jnp.float32(_THRESH_AFF))
    pltpu.sync_copy(kk_hbm.at[pl.ds(w * _L, _L)], vout.at[pl.ds(0, _L)])
    kv = vout[pl.ds(0, _L)]
    bufs = ((g_a, p_a, m_a), (g_b, p_b, m_b))
    sems = (sem_a, sem_b)

    def sweep(cand, want_final):
        def issue(t, slot):
            off = t * _CH
            return [
                pltpu.async_copy(gt_hbm.at[pl.ds(gbase + off, _CH)], bufs[slot][0], sems[slot]),
                pltpu.async_copy(pr_hbm.at[pl.ds(gbase + off, _CH)], bufs[slot][1], sems[slot]),
                pltpu.async_copy(mk_hbm.at[pl.ds(mbase + off, _CH)], bufs[slot][2], sems[slot]),
            ]

        zero = jnp.zeros((_L,), jnp.float32)
        accs = (zero, zero, zero)
        pending = [None, None]
        pending[0] = issue(0, 0)
        for t in range(_NCH_TOPK):
            slot = t % 2
            if t + 1 < _NCH_TOPK:
                pending[1 - slot] = issue(t + 1, 1 - slot)
            for h in pending[slot]:
                h.wait()
            buf = bufs[slot]

            def body(j, accs, buf=buf):
                cge, cgt, sgt = accs
                o = j * _L
                gt = buf[0][pl.ds(o, _L)]
                pr = buf[1][pl.ds(o, _L)]
                mk = buf[2][pl.ds(o, _L)]
                pos = gt > thr
                pr2 = jnp.where(pos & (pr > 1.0), 1.0, pr)
                d = pr2 - gt
                v = jnp.where(pos, jnp.float32(-1.0), d * d * mk)
                cge = cge + plsc.all_reduce_population_count(v >= cand).astype(jnp.float32)
                if want_final:
                    cgt = cgt + plsc.all_reduce_population_count(v > cand).astype(jnp.float32)
                    sgt = sgt + jnp.where(v > cand, v, jnp.float32(0.0))
                return (cge, cgt, sgt)

            accs = lax.fori_loop(0, _CH // _L, body, accs)
        return accs

    def phase_a(i, cand):
        cge, _, _ = sweep(cand, False)
        return jnp.where(cge >= kv, cand, cand * jnp.float32(1.0 / 65536.0))

    def phase_b(i, cand):
        c2 = cand * jnp.float32(2.0)
        cge, _, _ = sweep(c2, False)
        return jnp.where(cge >= kv, c2, cand)

    def phase_c(i, lohi):
        lo, hi = lohi
        mid = (lo + hi) * jnp.float32(0.5)
        cge, _, _ = sweep(mid, False)
        acc = cge >= kv
        return (jnp.where(acc, mid, lo), jnp.where(acc, hi, mid))

    start = jnp.full((_L,), 2.0**124, jnp.float32)
    cand = lax.fori_loop(0, 17, phase_a, start)
    cand = lax.fori_loop(0, 16, phase_b, cand)
    lo, hi = lax.fori_loop(0, 30, phase_c, (cand, cand * jnp.float32(2.0)))
    _, cgt, sgt = sweep(lo, True)
    vout[pl.ds(0, _L)] = sgt
    vout[pl.ds(_L, _L)] = cgt
    vout[pl.ds(2 * _L, _L)] = lo
    pltpu.sync_copy(vout, out_hbm.at[pl.ds(w * 3 * _L, 3 * _L)])


@functools.partial(
    pl.pallas_call,
    grid=(_TC_B,),
    in_specs=[pl.BlockSpec((1, _H, _W), lambda i: (i + _SC_B, 0, 0))] * 5,
    out_specs=pl.BlockSpec((1, 1, 8), lambda i: (i, 0, 0), memory_space=pltpu.SMEM),
    out_shape=jax.ShapeDtypeStruct((_TC_B, 1, 8), jnp.float32),
)
def _tc_stats(rg_ref, rp_ref, ag_ref, ap_ref, mk_ref, out_ref):
    rg = rg_ref[0]
    rp = rp_ref[0]
    ag = ag_ref[0]
    ap = ap_ref[0]
    mk = mk_ref[0]
    one = jnp.float32(1.0)
    zf = jnp.float32(0.0)
    pos_r = rg > _THRESH_REG
    dr = jnp.where(pos_r, jnp.minimum(rp, one), rp) - rg
    lr = dr * dr * mk
    pos_a = ag > _THRESH_AFF
    da = jnp.where(pos_a, jnp.minimum(ap, one), ap) - ag
    la = da * da * mk
    out_ref[0, 0, 0] = jnp.sum(pos_r.astype(jnp.float32))
    out_ref[0, 0, 1] = jnp.sum(jnp.where(pos_r, lr, zf))
    out_ref[0, 0, 2] = jnp.sum(lr)
    out_ref[0, 0, 3] = jnp.sum(pos_a.astype(jnp.float32))
    out_ref[0, 0, 4] = jnp.sum(jnp.where(pos_a, la, zf))
    out_ref[0, 0, 5] = jnp.sum(la)
    out_ref[0, 0, 6] = zf
    out_ref[0, 0, 7] = zf


def _combine(pos, psum, nsum, topk_mean):
    npix = jnp.float32(_N)
    neg = npix - pos
    posi = psum / jnp.maximum(pos, 1.0)
    nega_mean = jnp.where(neg > 0, nsum / jnp.maximum(neg, 1.0), 0.0)
    nega = jnp.where(neg < 3.0 * pos, nega_mean, topk_mean)
    return jnp.where(pos > 0, posi + nega, topk_mean)


def kernel(region_score_gt, affinity_score_gt, region_score_pred,
           affinity_score_pred, mask):
    stats = _stats_kernel(region_score_gt, region_score_pred,
                          affinity_score_gt, affinity_score_pred, mask)
    tc3 = _tc_stats(region_score_gt, region_score_pred,
                    affinity_score_gt, affinity_score_pred,
                    mask)
    tc = tc3[:, 0, :]
    per_w = stats.reshape(_NW, 6, _L)
    counts = per_w[:, (0, 3), 0]
    sums = per_w[:, (1, 2, 4, 5), :].sum(-1)
    sc_c = counts.reshape(_SC_B, _NW // _SC_B, 2).sum(1)
    sc_s = sums.reshape(_SC_B, _NW // _SC_B, 4).sum(1)
    pos_r = jnp.concatenate([sc_c[:, 0], tc[:, 0]])
    pos_a = jnp.concatenate([sc_c[:, 1], tc[:, 3]])
    psum_r = jnp.concatenate([sc_s[:, 0], tc[:, 1]])
    tsum_r = jnp.concatenate([sc_s[:, 1], tc[:, 2]])
    psum_a = jnp.concatenate([sc_s[:, 2], tc[:, 4]])
    tsum_a = jnp.concatenate([sc_s[:, 3], tc[:, 5]])
    nsum_r = tsum_r - psum_r
    nsum_a = tsum_a - psum_a

    npix = jnp.float32(_N)
    need = jnp.any((pos_r == 0) | (npix - pos_r >= 3.0 * pos_r)) | jnp.any(
        (pos_a == 0) | (npix - pos_a >= 3.0 * pos_a)
    )

    def rare_branch():
        rgf = region_score_gt.reshape(-1)
        agf = affinity_score_gt.reshape(-1)
        rpf = region_score_pred.reshape(-1)
        apf = affinity_score_pred.reshape(-1)
        mkf = mask.reshape(-1)
        gts = jnp.concatenate([rgf, agf])
        prs = jnp.concatenate([rpf, apf])
        k_r = jnp.where(pos_r > 0, 3.0 * pos_r, 500.0)
        k_a = jnp.where(pos_a > 0, 3.0 * pos_a, 500.0)
        kk2 = jnp.stack([k_r, k_a])
        kk = jnp.broadcast_to(kk2[:, :, None], (2, _B, _L)).reshape(-1)
        out = _topk_kernel(gts, prs, mkf, kk)
        o = out.reshape(2, _B, 3, _L)
        sgt = o[:, :, 0, :].sum(-1)
        cgt = o[:, :, 1, 0]
        tval = o[:, :, 2, 0]
        return (sgt + (kk2 - cgt) * tval) / kk2

    topk_means = lax.cond(need, rare_branch, lambda: jnp.zeros((2, _B), jnp.float32))

    contrib_r = _combine(pos_r, psum_r, nsum_r, topk_means[0])
    contrib_a = _combine(pos_a, psum_a, nsum_a, topk_means[1])
    char_loss = jnp.sum(contrib_r)
    affi_loss = jnp.sum(contrib_a)
    return _LAMBDA * char_loss / _B + affi_loss / _B
